# baseline (speedup 1.0000x reference)
import jax
import jax.numpy as jnp
from jax import lax
from jax.experimental import pallas as pl
from jax.experimental.pallas import tpu as pltpu
from jax.experimental.pallas import tpu_sc as plsc

N_ASSETS = 10
SUB_SIZE = 100000
TOTAL_VOCAB = N_ASSETS * SUB_SIZE
EMBED_DIM = 32
BATCH = 16384

_INFO = plsc.get_sparse_core_info()
_NC = _INFO.num_cores
_NS = _INFO.num_subcores
_LANES = _INFO.num_lanes
_NW = _NC * _NS
_BPW = BATCH // _NW
_NBLK = _BPW // _LANES

_LAST_TILE = (TOTAL_VOCAB // 128) - 1
_TAIL_START = (_LAST_TILE + 1) * 128
_TAIL_LEN = TOTAL_VOCAB - _TAIL_START


_G = 8
_NG = _BPW // _G


def _sc_body(asset_hbm, shape_hbm, tt_hbm, outt_hbm,
             asset_v, shape_v, idx_v, jc_v, slots_v, tail_v, rows_v,
             sem_a, sem_b, sem_c):
    wid = lax.axis_index("s") * _NC + lax.axis_index("c")
    base = wid * _BPW

    pltpu.sync_copy(asset_hbm.at[pl.ds(base, _BPW)], asset_v)
    pltpu.sync_copy(shape_hbm.at[pl.ds(base, _BPW)], shape_v)
    pltpu.sync_copy(tt_hbm.at[:, pl.ds(_TAIL_START, _TAIL_LEN)], tail_v)

    lane = lax.iota(jnp.int32, _LANES)

    for i in range(_NBLK):
        off = i * _LANES
        a = asset_v[pl.ds(off, _LANES)]
        s = shape_v[pl.ds(off, _LANES)]
        idx = a * SUB_SIZE + s
        idx_v[pl.ds(off, _LANES)] = idx
        jc_v[pl.ds(off, _LANES)] = jnp.minimum(
            lax.shift_right_logical(idx, 7), jnp.int32(_LAST_TILE))
    idx_v[pl.ds(_BPW, _LANES)] = lane * 0
    idx_v[pl.ds(_BPW + _LANES, _LANES)] = lane * 0
    jc_v[pl.ds(_BPW, _LANES)] = lane * 0
    jc_v[pl.ds(_BPW + _LANES, _LANES)] = lane * 0

    def _fire(g, slot_base, sem):
        jcs = jc_v[pl.ds(g * _G, _LANES)]
        for l in range(_G):
            wstart = pl.multiple_of(jcs[l] * 128, 128)
            pltpu.make_async_copy(
                tt_hbm.at[:, pl.ds(wstart, 128)],
                slots_v.at[slot_base + l],
                sem,
            ).start()

    def _drain(slot_base, sem):
        for l in range(_G):
            pltpu.make_async_copy(
                tt_hbm.at[:, pl.ds(0, 128)],
                slots_v.at[slot_base + l],
                sem,
            ).wait()

    def _extract(g, slot_base):
        vec = idx_v[pl.ds(g * _G, _LANES)]
        for l in range(_G):
            i = vec[l]
            cvec = lane * 0 + (i & 127)
            jvec = lane * 0 + (g * _G + l)
            is_tail = i >= _TAIL_START
            trow = lane * 0 + jnp.maximum(i - _TAIL_START, 0)
            slot = slots_v.at[slot_base + l]
            for h in range(EMBED_DIM // _LANES):
                e16 = h * _LANES + lane
                vm = plsc.load_gather(slot, [e16, cvec])
                vt = plsc.load_gather(tail_v, [e16, trow])
                val = jnp.where(is_tail, vt, vm)
                plsc.store_scatter(rows_v, [e16, jvec], val)

    _fire(0, 0, sem_a)
    _fire(1, _G, sem_b)
    _fire(2, 2 * _G, sem_c)

    def _triple(t, _):
        g = 3 * t
        for ph, sem in ((0, sem_a), (1, sem_b), (2, sem_c)):
            _drain(ph * _G, sem)
            _extract(g + ph, ph * _G)

            @pl.when(g + ph + 3 < _NG)
            def _(ph=ph, sem=sem):
                _fire(g + ph + 3, ph * _G, sem)

        return 0

    lax.fori_loop(0, _NG // 3, _triple, 0)

    for g in range((_NG // 3) * 3, _NG):
        ph = g % 3
        sem = (sem_a, sem_b, sem_c)[ph]
        _drain(ph * _G, sem)
        _extract(g, ph * _G)

    pltpu.sync_copy(rows_v, outt_hbm.at[:, pl.ds(base, _BPW)])


@jax.jit
def _lookup(asset_index, shape_index, table_t):
    mesh = plsc.VectorSubcoreMesh(core_axis_name="c", subcore_axis_name="s")
    fn = pl.kernel(
        _sc_body,
        out_type=jax.ShapeDtypeStruct((EMBED_DIM, BATCH), jnp.float32),
        mesh=mesh,
        scratch_types=[
            pltpu.VMEM((_BPW,), jnp.int32),
            pltpu.VMEM((_BPW,), jnp.int32),
            pltpu.VMEM((_BPW + 2 * _LANES,), jnp.int32),
            pltpu.VMEM((_BPW + 2 * _LANES,), jnp.int32),
            pltpu.VMEM((3 * _G, EMBED_DIM, 128), jnp.float32),
            pltpu.VMEM((EMBED_DIM, _TAIL_LEN), jnp.float32),
            pltpu.VMEM((EMBED_DIM, _BPW), jnp.float32),
            pltpu.SemaphoreType.DMA,
            pltpu.SemaphoreType.DMA,
            pltpu.SemaphoreType.DMA,
        ],
        compiler_params=pltpu.CompilerParams(needs_layout_passes=False),
    )
    return fn(asset_index, shape_index, table_t)


def kernel(asset_index, shape_index, table):
    out_t = _lookup(asset_index.astype(jnp.int32),
                    shape_index.astype(jnp.int32),
                    table.T)
    return out_t.T
